# baseline (speedup 1.0000x reference)
import functools

import jax
import jax.numpy as jnp
from jax import lax
from jax.experimental import pallas as pl
from jax.experimental.pallas import tpu as pltpu
from jax.experimental.pallas import tpu_sc as plsc

NUM_LATTICES = 1024
LATTICE_RANK = 4
NUM_INPUTS = 128
BATCH = 4096
LANES = 16

NUM_CORES = 2
NUM_SUBCORES = 16
NW = NUM_CORES * NUM_SUBCORES
TB = BATCH // NW
LCHUNK = 256
NCHUNK = NUM_LATTICES // LCHUNK
GROUPS_PER_CHUNK = LCHUNK // LANES


def _tec_body(xtt_hbm, kt_hbm, idxt_hbm, out_hbm, xv, kv, iv, outv):
    wid = lax.axis_index("s") * NUM_CORES + lax.axis_index("c")

    pltpu.sync_copy(xtt_hbm.at[wid], xv.at[:, pl.ds(0, TB)])
    pltpu.sync_copy(kt_hbm, kv)
    pltpu.sync_copy(idxt_hbm, iv)

    def clip_row(r, carry):
        for j in range(NUM_INPUTS // LANES):
            v = xv[r, pl.ds(j * LANES, LANES)]
            xv[r, pl.ds(j * LANES, LANES)] = jnp.minimum(
                jnp.maximum(v, 0.0), 1.0)
        return carry

    lax.fori_loop(0, NUM_INPUTS, clip_row, 0, unroll=2)

    for c in range(NCHUNK):
        def group_body(lg, carry, c=c):
            g16 = (c * GROUPS_PER_CHUNK + lg) * LANES
            iv0 = iv[0, pl.ds(g16, LANES)]
            iv1 = iv[1, pl.ds(g16, LANES)]
            iv2 = iv[2, pl.ds(g16, LANES)]
            iv3 = iv[3, pl.ds(g16, LANES)]
            e = [kv[j, pl.ds(g16, LANES)] for j in range(8)]
            d = [kv[j + 8, pl.ds(g16, LANES)] - e[j] for j in range(8)]

            def row_body(b, lg=lg):
                bvec = jnp.full((LANES,), b, jnp.int32)
                x0 = plsc.load_gather(xv, [iv0, bvec])
                x1 = plsc.load_gather(xv, [iv1, bvec])
                x2 = plsc.load_gather(xv, [iv2, bvec])
                x3 = plsc.load_gather(xv, [iv3, bvec])
                tA = [e[j] + d[j] * x0 for j in range(8)]
                tB = [tA[j] + (tA[j + 4] - tA[j]) * x1 for j in range(4)]
                tC = [tB[j] + (tB[j + 2] - tB[j]) * x2 for j in range(2)]
                res = tC[0] + (tC[1] - tC[0]) * x3
                outv[b, pl.ds(lg * LANES, LANES)] = res

            plsc.parallel_loop(0, TB, unroll=8)(row_body)
            return carry

        lax.fori_loop(0, GROUPS_PER_CHUNK, group_body, 0)
        pltpu.sync_copy(
            outv,
            out_hbm.at[pl.ds(wid * TB, TB), pl.ds(c * LCHUNK, LCHUNK)])


@functools.partial(jax.jit, static_argnames=())
def _rtl_sc(xtt, kt, idxt):
    mesh = plsc.VectorSubcoreMesh(
        core_axis_name="c", subcore_axis_name="s")
    run = pl.kernel(
        _tec_body,
        out_type=jax.ShapeDtypeStruct((BATCH, NUM_LATTICES), jnp.float32),
        mesh=mesh,
        scratch_types=[
            pltpu.VMEM((NUM_INPUTS, TB + 1), jnp.float32),
            pltpu.VMEM((LANES, NUM_LATTICES), jnp.float32),
            pltpu.VMEM((LATTICE_RANK, NUM_LATTICES), jnp.int32),
            pltpu.VMEM((TB, LCHUNK), jnp.float32),
        ],
        compiler_params=pltpu.CompilerParams(needs_layout_passes=False),
    )
    return run(xtt, kt, idxt)


def kernel(x, lattice_indices, kernel):
    xtt = x.reshape(NW, TB, NUM_INPUTS).transpose(0, 2, 1)
    kt = kernel.T
    idxt = lattice_indices.T.astype(jnp.int32)
    return _rtl_sc(xtt, kt, idxt)


# speedup vs baseline: 1.1312x; 1.1312x over previous
import functools

import jax
import jax.numpy as jnp
from jax import lax
from jax.experimental import pallas as pl
from jax.experimental.pallas import tpu as pltpu
from jax.experimental.pallas import tpu_sc as plsc

NUM_LATTICES = 1024
LATTICE_RANK = 4
NUM_INPUTS = 128
BATCH = 4096
LANES = 16

NUM_CORES = 2
NUM_SUBCORES = 16
NW = NUM_CORES * NUM_SUBCORES
TB = BATCH // NW
LCHUNK = 256
NCHUNK = NUM_LATTICES // LCHUNK
GROUPS_PER_CHUNK = LCHUNK // LANES


def _tec_body(xtt_hbm, kt_hbm, idxt_hbm, out_hbm, xv, kv, iv, outv):
    wid = lax.axis_index("s") * NUM_CORES + lax.axis_index("c")

    pltpu.sync_copy(xtt_hbm.at[wid], xv)
    pltpu.sync_copy(kt_hbm, kv)
    pltpu.sync_copy(idxt_hbm, iv)

    def clip_row(r, carry):
        for j in range(NUM_INPUTS // LANES):
            v = xv[r, pl.ds(j * LANES, LANES)]
            xv[r, pl.ds(j * LANES, LANES)] = jnp.minimum(
                jnp.maximum(v, 0.0), 1.0)
        return carry

    lax.fori_loop(0, NUM_INPUTS, clip_row, 0, unroll=2)

    for c in range(NCHUNK):
        def group_body(lg, carry, c=c):
            g16 = (c * GROUPS_PER_CHUNK + lg) * LANES
            iv0 = iv[0, pl.ds(g16, LANES)]
            iv1 = iv[1, pl.ds(g16, LANES)]
            iv2 = iv[2, pl.ds(g16, LANES)]
            iv3 = iv[3, pl.ds(g16, LANES)]
            cf = [kv[j, pl.ds(g16, LANES)] for j in range(16)]
            for dlev in (8, 4, 2, 1):
                cf = [cf[j] for j in range(16)]
                for j in range(16):
                    if j & dlev:
                        cf[j] = cf[j] - cf[j ^ dlev]

            def row_body(b, bvec, lg=lg):
                x0 = plsc.load_gather(xv, [iv0, bvec])
                x1 = plsc.load_gather(xv, [iv1, bvec])
                x2 = plsc.load_gather(xv, [iv2, bvec])
                x3 = plsc.load_gather(xv, [iv3, bvec])
                tA = [cf[j] + cf[j + 8] * x0 for j in range(8)]
                tB = [tA[j] + tA[j + 4] * x1 for j in range(4)]
                tC = [tB[j] + tB[j + 2] * x2 for j in range(2)]
                res = tC[0] + tC[1] * x3
                outv[b, pl.ds(lg * LANES, LANES)] = res
                return bvec + 1

            plsc.parallel_loop(
                0, TB, unroll=4,
                carry=jnp.zeros((LANES,), jnp.int32),
            )(row_body)
            return carry

        lax.fori_loop(0, GROUPS_PER_CHUNK, group_body, 0)
        pltpu.sync_copy(
            outv,
            out_hbm.at[pl.ds(wid * TB, TB), pl.ds(c * LCHUNK, LCHUNK)])


@functools.partial(jax.jit, static_argnames=())
def _rtl_sc(xtt, kt, idxt):
    mesh = plsc.VectorSubcoreMesh(
        core_axis_name="c", subcore_axis_name="s")
    run = pl.kernel(
        _tec_body,
        out_type=jax.ShapeDtypeStruct((BATCH, NUM_LATTICES), jnp.float32),
        mesh=mesh,
        scratch_types=[
            pltpu.VMEM((NUM_INPUTS, TB), jnp.float32),
            pltpu.VMEM((LANES, NUM_LATTICES), jnp.float32),
            pltpu.VMEM((LATTICE_RANK, NUM_LATTICES), jnp.int32),
            pltpu.VMEM((TB, LCHUNK), jnp.float32),
        ],
        compiler_params=pltpu.CompilerParams(needs_layout_passes=False),
    )
    return run(xtt, kt, idxt)


def kernel(x, lattice_indices, kernel):
    xtt = x.reshape(NW, TB, NUM_INPUTS).transpose(0, 2, 1)
    kt = kernel.T
    idxt = lattice_indices.T.astype(jnp.int32)
    return _rtl_sc(xtt, kt, idxt)


# speedup vs baseline: 1.1407x; 1.0084x over previous
import functools

import jax
import jax.numpy as jnp
from jax import lax
from jax.experimental import pallas as pl
from jax.experimental.pallas import tpu as pltpu
from jax.experimental.pallas import tpu_sc as plsc

NUM_LATTICES = 1024
LATTICE_RANK = 4
NUM_INPUTS = 128
BATCH = 4096
LANES = 16

NUM_CORES = 2
NUM_SUBCORES = 16
NW = NUM_CORES * NUM_SUBCORES
TB = BATCH // NW
LCHUNK = 256
NCHUNK = NUM_LATTICES // LCHUNK
GROUPS_PER_CHUNK = LCHUNK // LANES


def _tec_body(xtt_hbm, kt_hbm, idxt_hbm, out_hbm, xv, kv, iv, outv):
    wid = lax.axis_index("s") * NUM_CORES + lax.axis_index("c")

    pltpu.sync_copy(xtt_hbm.at[wid], xv)
    pltpu.sync_copy(kt_hbm, kv)
    pltpu.sync_copy(idxt_hbm, iv)

    def clip_row(r, carry):
        for j in range(NUM_INPUTS // LANES):
            v = xv[r, pl.ds(j * LANES, LANES)]
            xv[r, pl.ds(j * LANES, LANES)] = jnp.minimum(
                jnp.maximum(v, 0.0), 1.0)
        return carry

    lax.fori_loop(0, NUM_INPUTS, clip_row, 0, unroll=2)

    for c in range(NCHUNK):
        def group_body(lg, carry, c=c):
            g16 = (c * GROUPS_PER_CHUNK + lg) * LANES
            iv0 = iv[0, pl.ds(g16, LANES)]
            iv1 = iv[1, pl.ds(g16, LANES)]
            iv2 = iv[2, pl.ds(g16, LANES)]
            iv3 = iv[3, pl.ds(g16, LANES)]
            cf = [kv[j, pl.ds(g16, LANES)] for j in range(16)]
            for dlev in (8, 4, 2, 1):
                cf = [cf[j] for j in range(16)]
                for j in range(16):
                    if j & dlev:
                        cf[j] = cf[j] - cf[j ^ dlev]

            def row_body(b, lg=lg):
                bvec = jnp.full((LANES,), b, jnp.int32)
                x0 = plsc.load_gather(xv, [iv0, bvec])
                x1 = plsc.load_gather(xv, [iv1, bvec])
                x2 = plsc.load_gather(xv, [iv2, bvec])
                x3 = plsc.load_gather(xv, [iv3, bvec])
                tA = [cf[j] + cf[j + 8] * x0 for j in range(8)]
                tB = [tA[j] + tA[j + 4] * x1 for j in range(4)]
                tC = [tB[j] + tB[j + 2] * x2 for j in range(2)]
                res = tC[0] + tC[1] * x3
                outv[b, pl.ds(lg * LANES, LANES)] = res

            plsc.parallel_loop(0, TB, unroll=4)(row_body)
            return carry

        lax.fori_loop(0, GROUPS_PER_CHUNK, group_body, 0)
        pltpu.sync_copy(
            outv,
            out_hbm.at[pl.ds(wid * TB, TB), pl.ds(c * LCHUNK, LCHUNK)])


@functools.partial(jax.jit, static_argnames=())
def _rtl_sc(xtt, kt, idxt):
    mesh = plsc.VectorSubcoreMesh(
        core_axis_name="c", subcore_axis_name="s")
    run = pl.kernel(
        _tec_body,
        out_type=jax.ShapeDtypeStruct((BATCH, NUM_LATTICES), jnp.float32),
        mesh=mesh,
        scratch_types=[
            pltpu.VMEM((NUM_INPUTS, TB), jnp.float32),
            pltpu.VMEM((LANES, NUM_LATTICES), jnp.float32),
            pltpu.VMEM((LATTICE_RANK, NUM_LATTICES), jnp.int32),
            pltpu.VMEM((TB, LCHUNK), jnp.float32),
        ],
        compiler_params=pltpu.CompilerParams(needs_layout_passes=False),
    )
    return run(xtt, kt, idxt)


def kernel(x, lattice_indices, kernel):
    xtt = x.reshape(NW, TB, NUM_INPUTS).transpose(0, 2, 1)
    kt = kernel.T
    idxt = lattice_indices.T.astype(jnp.int32)
    return _rtl_sc(xtt, kt, idxt)


# speedup vs baseline: 2.6016x; 2.2808x over previous
import functools

import jax
import jax.numpy as jnp
from jax import lax
from jax.experimental import pallas as pl
from jax.experimental.pallas import tpu as pltpu
from jax.experimental.pallas import tpu_sc as plsc

NUM_LATTICES = 1024
LATTICE_RANK = 4
NUM_INPUTS = 128
BATCH = 4096
LANES = 16

NUM_CORES = 2
NUM_SUBCORES = 16
NW = NUM_CORES * NUM_SUBCORES
TB = BATCH // NW
BGROUPS = TB // LANES
LCHUNK = 256
NCHUNK = NUM_LATTICES // LCHUNK
NGROUPS = NUM_LATTICES // LANES


def _tec_body(xtt_hbm, kt_hbm, idx_hbm, out_hbm, xv, kv, iv, outv):
    wid = lax.axis_index("s") * NUM_CORES + lax.axis_index("c")

    pltpu.sync_copy(xtt_hbm.at[wid], xv)
    pltpu.sync_copy(kt_hbm, kv)
    pltpu.sync_copy(idx_hbm, iv.at[pl.ds(0, NUM_LATTICES * LATTICE_RANK)])

    def clip_row(r):
        for j in range(NUM_INPUTS // LANES):
            v = xv[r, pl.ds(j * LANES, LANES)]
            xv[r, pl.ds(j * LANES, LANES)] = jnp.minimum(
                jnp.maximum(v, 0.0), 1.0)

    plsc.parallel_loop(0, NUM_INPUTS, unroll=2)(clip_row)

    def mobius_group(g):
        g16 = g * LANES
        cf = [kv[j, pl.ds(g16, LANES)] for j in range(16)]
        for dlev in (8, 4, 2, 1):
            for j in range(16):
                if j & dlev:
                    cf[j] = cf[j] - cf[j ^ dlev]
        for j in range(16):
            kv[j, pl.ds(g16, LANES)] = cf[j]

    plsc.parallel_loop(0, NGROUPS, unroll=1)(mobius_group)

    jvec = lax.broadcasted_iota(jnp.int32, (LANES,), 0)

    def splat(vec, j):
        return lax.gather(
            vec, jnp.full((LANES, 1), j, jnp.int32),
            lax.GatherDimensionNumbers(
                offset_dims=(), collapsed_slice_dims=(0,),
                start_index_map=(0,)),
            (1,), mode=lax.GatherScatterMode.PROMISE_IN_BOUNDS)

    for c in range(NCHUNK):
        def lat_body(ll, c=c):
            l = c * LCHUNK + ll
            lvec = jnp.full((LANES,), l, jnp.int32)
            cfl = plsc.load_gather(kv, [jvec, lvec])
            u = [splat(cfl, j) for j in range(16)]
            ivv = iv[pl.ds(l * LATTICE_RANK, LANES)]
            i0 = ivv[0]
            i1 = ivv[1]
            i2 = ivv[2]
            i3 = ivv[3]
            for gb in range(BGROUPS):
                bs = pl.ds(gb * LANES, LANES)
                x0 = xv[i0, bs]
                x1 = xv[i1, bs]
                x2 = xv[i2, bs]
                x3 = xv[i3, bs]
                tA = [u[j] + u[j + 8] * x0 for j in range(8)]
                tB = [tA[j] + tA[j + 4] * x1 for j in range(4)]
                tC = [tB[j] + tB[j + 2] * x2 for j in range(2)]
                outv[ll, bs] = tC[0] + tC[1] * x3

        plsc.parallel_loop(0, LCHUNK, unroll=1)(lat_body)
        pltpu.sync_copy(
            outv,
            out_hbm.at[pl.ds(c * LCHUNK, LCHUNK), pl.ds(wid * TB, TB)])


@functools.partial(jax.jit, static_argnames=())
def _rtl_sc(xtt, kt, idx):
    mesh = plsc.VectorSubcoreMesh(
        core_axis_name="c", subcore_axis_name="s")
    run = pl.kernel(
        _tec_body,
        out_type=jax.ShapeDtypeStruct((NUM_LATTICES, BATCH), jnp.float32),
        mesh=mesh,
        scratch_types=[
            pltpu.VMEM((NUM_INPUTS, TB), jnp.float32),
            pltpu.VMEM((LANES, NUM_LATTICES), jnp.float32),
            pltpu.VMEM((NUM_LATTICES * LATTICE_RANK + LANES,),
                       jnp.int32),
            pltpu.VMEM((LCHUNK, TB), jnp.float32),
        ],
        compiler_params=pltpu.CompilerParams(needs_layout_passes=False),
    )
    return run(xtt, kt, idx)


def kernel(x, lattice_indices, kernel):
    xtt = x.reshape(NW, TB, NUM_INPUTS).transpose(0, 2, 1)
    kt = kernel.T
    idx = lattice_indices.astype(jnp.int32).reshape(-1)
    return _rtl_sc(xtt, kt, idx).T
